# baseline (speedup 1.0000x reference)
import jax
import jax.numpy as jnp
from jax import lax
from jax.experimental import pallas as pl
from jax.experimental.pallas import tpu as pltpu
from jax.experimental.pallas import tpu_sc as plsc

N_NODES = 10000
D_IN = 128
DEG_W = 16
N_EDGES = 320000

NC = 2
NS = 16
NW = NC * NS
EDGES_PER_TILE = N_EDGES // NW
CHUNK = 125
NSTEPS = EDGES_PER_TILE // CHUNK
NH = NSTEPS // 2
SLAB = N_NODES // NS
H1 = 64
H2 = CHUNK - H1


def _sc_body(x_hbm, src_hbm, dst_hbm, zf_hbm, zd_hbm, ones_hbm,
             agg_hbm, deg_hbm,
             sa, da, sb, db, buf_a, buf_b, ones_v, acc_sh, deg_sh,
             sem_ia, sem_ib, sem_g1, sem_g2, sem_h1, sem_h2,
             sem_s1, sem_s2, sem_s3):
    c = lax.axis_index("c")
    s = lax.axis_index("s")

    def gissue(r, buf, s1, s2):
        pltpu.async_copy(x_hbm.at[r.at[pl.ds(0, H1)]], buf.at[pl.ds(0, H1)], s1)
        pltpu.async_copy(x_hbm.at[r.at[pl.ds(H1, H2)]], buf.at[pl.ds(H1, H2)], s2)

    def gwait(r, buf, s1, s2):
        pltpu.make_async_copy(
            x_hbm.at[r.at[pl.ds(0, H1)]], buf.at[pl.ds(0, H1)], s1).wait()
        pltpu.make_async_copy(
            x_hbm.at[r.at[pl.ds(H1, H2)]], buf.at[pl.ds(H1, H2)], s2).wait()

    def iissue(e, sr, dr, sem):
        pltpu.async_copy(src_hbm.at[c, s, e], sr, sem)
        pltpu.async_copy(dst_hbm.at[c, s, e], dr, sem)

    def iwait(e, sr, dr, sem):
        pltpu.make_async_copy(src_hbm.at[c, s, e], sr, sem).wait()
        pltpu.make_async_copy(dst_hbm.at[c, s, e], dr, sem).wait()

    def scat(buf, dr):
        pltpu.async_copy(buf.at[pl.ds(0, H1)],
                         acc_sh.at[dr.at[pl.ds(0, H1)]], sem_s1, add=True)
        pltpu.async_copy(buf.at[pl.ds(H1, H2)],
                         acc_sh.at[dr.at[pl.ds(H1, H2)]], sem_s2, add=True)
        pltpu.async_copy(ones_v, deg_sh.at[dr], sem_s3, add=True)
        pltpu.make_async_copy(buf.at[pl.ds(0, H1)],
                              acc_sh.at[dr.at[pl.ds(0, H1)]], sem_s1).wait()
        pltpu.make_async_copy(buf.at[pl.ds(H1, H2)],
                              acc_sh.at[dr.at[pl.ds(H1, H2)]], sem_s2).wait()
        pltpu.make_async_copy(ones_v, deg_sh.at[dr], sem_s3).wait()

    slab = pl.ds(s * SLAB, SLAB)
    pltpu.sync_copy(zf_hbm.at[slab], acc_sh.at[slab])
    pltpu.sync_copy(zd_hbm.at[slab], deg_sh.at[slab])
    pltpu.sync_copy(ones_hbm, ones_v)
    pltpu.sync_copy(src_hbm.at[c, s, 0], sa)
    pltpu.sync_copy(dst_hbm.at[c, s, 0], da)
    plsc.subcore_barrier()
    gissue(sa, buf_a, sem_g1, sem_g2)
    iissue(1, sb, db, sem_ib)

    def step(j, carry):
        iwait(2 * j + 1, sb, db, sem_ib)
        gissue(sb, buf_b, sem_h1, sem_h2)
        gwait(sa, buf_a, sem_g1, sem_g2)
        scat(buf_a, da)

        @pl.when(j < NH - 1)
        def _():
            iissue(2 * j + 2, sa, da, sem_ia)

        gwait(sb, buf_b, sem_h1, sem_h2)
        scat(buf_b, db)

        @pl.when(j < NH - 1)
        def _():
            iwait(2 * j + 2, sa, da, sem_ia)
            gissue(sa, buf_a, sem_g1, sem_g2)
            iissue(2 * j + 3, sb, db, sem_ib)

        return carry

    lax.fori_loop(0, NH, step, 0)
    plsc.subcore_barrier()

    pltpu.sync_copy(acc_sh.at[slab], agg_hbm.at[c, slab])
    pltpu.sync_copy(deg_sh.at[slab], deg_hbm.at[c, slab])


def _sc_aggregate(x, src4, dst4, zf, zd, ones):
    mesh = plsc.VectorSubcoreMesh(core_axis_name="c", subcore_axis_name="s")
    fn = pl.kernel(
        _sc_body,
        out_type=(
            jax.ShapeDtypeStruct((NC, N_NODES, D_IN), jnp.float32),
            jax.ShapeDtypeStruct((NC, N_NODES, DEG_W), jnp.float32),
        ),
        mesh=mesh,
        scratch_types=[
            pltpu.VMEM((CHUNK,), jnp.int32),
            pltpu.VMEM((CHUNK,), jnp.int32),
            pltpu.VMEM((CHUNK,), jnp.int32),
            pltpu.VMEM((CHUNK,), jnp.int32),
            pltpu.VMEM((CHUNK, D_IN), jnp.float32),
            pltpu.VMEM((CHUNK, D_IN), jnp.float32),
            pltpu.VMEM((CHUNK, DEG_W), jnp.float32),
            pltpu.VMEM_SHARED((N_NODES, D_IN), jnp.float32),
            pltpu.VMEM_SHARED((N_NODES, DEG_W), jnp.float32),
        ] + [pltpu.SemaphoreType.DMA] * 9,
        compiler_params=pltpu.CompilerParams(use_tc_tiling_on_sc=False),
    )
    return fn(x, src4, dst4, zf, zd, ones)


def _tc_body(x_ref, p_ref, d_ref, wlt_ref, wrt_ref, b_ref, o_ref):
    p = p_ref[...]
    d = d_ref[...]
    agg = p[0] + p[1]
    deg = d[0, :, 0:1] + d[1, :, 0:1]
    mean = agg / jnp.maximum(deg, 1.0)
    out = (jnp.dot(mean, wlt_ref[...], preferred_element_type=jnp.float32)
           + jnp.dot(x_ref[...], wrt_ref[...], preferred_element_type=jnp.float32)
           + b_ref[...])
    o_ref[...] = jnp.maximum(out, 0.0)


def _tc_combine(x, agg, deg, wlt, wrt, b2):
    blk = 1000
    grid = N_NODES // blk
    return pl.pallas_call(
        _tc_body,
        grid=(grid,),
        in_specs=[
            pl.BlockSpec((blk, D_IN), lambda i: (i, 0)),
            pl.BlockSpec((NC, blk, D_IN), lambda i: (0, i, 0)),
            pl.BlockSpec((NC, blk, DEG_W), lambda i: (0, i, 0)),
            pl.BlockSpec((D_IN, D_IN), lambda i: (0, 0)),
            pl.BlockSpec((D_IN, D_IN), lambda i: (0, 0)),
            pl.BlockSpec((1, D_IN), lambda i: (0, 0)),
        ],
        out_specs=pl.BlockSpec((blk, D_IN), lambda i: (i, 0)),
        out_shape=jax.ShapeDtypeStruct((N_NODES, D_IN), jnp.float32),
    )(x, agg, deg, wlt, wrt, b2)


@jax.jit
def kernel(x, edge_index, W_l, W_r, b):
    src4 = edge_index[0].astype(jnp.int32).reshape(NC, NS, NSTEPS, CHUNK)
    dst4 = edge_index[1].astype(jnp.int32).reshape(NC, NS, NSTEPS, CHUNK)
    zf = jnp.zeros((N_NODES, D_IN), jnp.float32)
    zd = jnp.zeros((N_NODES, DEG_W), jnp.float32)
    ones = jnp.ones((CHUNK, DEG_W), jnp.float32)
    agg, deg = _sc_aggregate(x, src4, dst4, zf, zd, ones)
    return _tc_combine(x, agg, deg, W_l.T, W_r.T, b[None, :])


# speedup vs baseline: 1.0103x; 1.0103x over previous
import jax
import jax.numpy as jnp
from jax import lax
from jax.experimental import pallas as pl
from jax.experimental.pallas import tpu as pltpu
from jax.experimental.pallas import tpu_sc as plsc

N_NODES = 10000
D_IN = 128
DEG_W = 16
N_EDGES = 320000

NC = 2
NS = 16
NW = NC * NS
EDGES_PER_TILE = N_EDGES // NW
CHUNK = 125
NSTEPS = EDGES_PER_TILE // CHUNK
NH = NSTEPS // 2
H1 = 64
H2 = CHUNK - H1


def _sc_body(x_hbm, src_hbm, dst_hbm, zf_hbm, zd_hbm, ones_hbm,
             agg_hbm, deg_hbm,
             sa, da, sb, db, buf_a, buf_b, ones_v, acc_sh, deg_sh,
             sem_ia, sem_ib, sem_g1, sem_g2, sem_h1, sem_h2,
             sem_s1, sem_s2, sem_s3):
    c = lax.axis_index("c")
    s = lax.axis_index("s")

    def gissue(r, buf, s1, s2):
        pltpu.async_copy(x_hbm.at[r.at[pl.ds(0, 48)]], buf.at[pl.ds(0, 48)], s1)
        pltpu.async_copy(x_hbm.at[r.at[pl.ds(48, 48)]], buf.at[pl.ds(48, 48)], s2)
        pltpu.async_copy(x_hbm.at[r.at[pl.ds(96, 29)]], buf.at[pl.ds(96, 29)], s1)

    def gwait(r, buf, s1, s2):
        pltpu.make_async_copy(
            x_hbm.at[r.at[pl.ds(0, 48)]], buf.at[pl.ds(0, 48)], s1).wait()
        pltpu.make_async_copy(
            x_hbm.at[r.at[pl.ds(48, 48)]], buf.at[pl.ds(48, 48)], s2).wait()
        pltpu.make_async_copy(
            x_hbm.at[r.at[pl.ds(96, 29)]], buf.at[pl.ds(96, 29)], s1).wait()

    def iissue(e, sr, dr, sem):
        pltpu.async_copy(src_hbm.at[c, s, e], sr, sem)
        pltpu.async_copy(dst_hbm.at[c, s, e], dr, sem)

    def iwait(e, sr, dr, sem):
        pltpu.make_async_copy(src_hbm.at[c, s, e], sr, sem).wait()
        pltpu.make_async_copy(dst_hbm.at[c, s, e], dr, sem).wait()

    def scat(buf, dr):
        pltpu.async_copy(buf.at[pl.ds(0, H1)],
                         acc_sh.at[dr.at[pl.ds(0, H1)]], sem_s1, add=True)
        pltpu.async_copy(buf.at[pl.ds(H1, H2)],
                         acc_sh.at[dr.at[pl.ds(H1, H2)]], sem_s2, add=True)
        pltpu.async_copy(ones_v, deg_sh.at[dr], sem_s3, add=True)
        pltpu.make_async_copy(buf.at[pl.ds(0, H1)],
                              acc_sh.at[dr.at[pl.ds(0, H1)]], sem_s1).wait()
        pltpu.make_async_copy(buf.at[pl.ds(H1, H2)],
                              acc_sh.at[dr.at[pl.ds(H1, H2)]], sem_s2).wait()
        pltpu.make_async_copy(ones_v, deg_sh.at[dr], sem_s3).wait()

    @pl.when(s == 0)
    def _():
        pltpu.sync_copy(zf_hbm, acc_sh)

    @pl.when(s == 1)
    def _():
        pltpu.sync_copy(zd_hbm, deg_sh)

    pltpu.sync_copy(ones_hbm, ones_v)
    pltpu.sync_copy(src_hbm.at[c, s, 0], sa)
    pltpu.sync_copy(dst_hbm.at[c, s, 0], da)
    plsc.subcore_barrier()
    gissue(sa, buf_a, sem_g1, sem_g2)
    iissue(1, sb, db, sem_ib)

    def step(j, carry):
        iwait(2 * j + 1, sb, db, sem_ib)
        gissue(sb, buf_b, sem_h1, sem_h2)
        gwait(sa, buf_a, sem_g1, sem_g2)
        scat(buf_a, da)

        @pl.when(j < NH - 1)
        def _():
            iissue(2 * j + 2, sa, da, sem_ia)

        gwait(sb, buf_b, sem_h1, sem_h2)
        scat(buf_b, db)

        @pl.when(j < NH - 1)
        def _():
            iwait(2 * j + 2, sa, da, sem_ia)
            gissue(sa, buf_a, sem_g1, sem_g2)
            iissue(2 * j + 3, sb, db, sem_ib)

        return carry

    lax.fori_loop(0, NH, step, 0)
    plsc.subcore_barrier()

    @pl.when(s == 0)
    def _():
        pltpu.sync_copy(acc_sh, agg_hbm.at[c])

    @pl.when(s == 1)
    def _():
        pltpu.sync_copy(deg_sh, deg_hbm.at[c])


def _sc_aggregate(x, src4, dst4, zf, zd, ones):
    mesh = plsc.VectorSubcoreMesh(core_axis_name="c", subcore_axis_name="s")
    fn = pl.kernel(
        _sc_body,
        out_type=(
            jax.ShapeDtypeStruct((NC, N_NODES, D_IN), jnp.float32),
            jax.ShapeDtypeStruct((NC, N_NODES, DEG_W), jnp.float32),
        ),
        mesh=mesh,
        scratch_types=[
            pltpu.VMEM((CHUNK,), jnp.int32),
            pltpu.VMEM((CHUNK,), jnp.int32),
            pltpu.VMEM((CHUNK,), jnp.int32),
            pltpu.VMEM((CHUNK,), jnp.int32),
            pltpu.VMEM((CHUNK, D_IN), jnp.float32),
            pltpu.VMEM((CHUNK, D_IN), jnp.float32),
            pltpu.VMEM((CHUNK, DEG_W), jnp.float32),
            pltpu.VMEM_SHARED((N_NODES, D_IN), jnp.float32),
            pltpu.VMEM_SHARED((N_NODES, DEG_W), jnp.float32),
        ] + [pltpu.SemaphoreType.DMA] * 9,
        compiler_params=pltpu.CompilerParams(use_tc_tiling_on_sc=False),
    )
    return fn(x, src4, dst4, zf, zd, ones)


def _tc_body(x_ref, p_ref, d_ref, wlt_ref, wrt_ref, b_ref, o_ref):
    p = p_ref[...]
    d = d_ref[...]
    agg = p[0] + p[1]
    deg = d[0, :, 0:1] + d[1, :, 0:1]
    mean = agg / jnp.maximum(deg, 1.0)
    out = (jnp.dot(mean, wlt_ref[...], preferred_element_type=jnp.float32)
           + jnp.dot(x_ref[...], wrt_ref[...], preferred_element_type=jnp.float32)
           + b_ref[...])
    o_ref[...] = jnp.maximum(out, 0.0)


def _tc_combine(x, agg, deg, wlt, wrt, b2):
    blk = 1000
    grid = N_NODES // blk
    return pl.pallas_call(
        _tc_body,
        grid=(grid,),
        in_specs=[
            pl.BlockSpec((blk, D_IN), lambda i: (i, 0)),
            pl.BlockSpec((NC, blk, D_IN), lambda i: (0, i, 0)),
            pl.BlockSpec((NC, blk, DEG_W), lambda i: (0, i, 0)),
            pl.BlockSpec((D_IN, D_IN), lambda i: (0, 0)),
            pl.BlockSpec((D_IN, D_IN), lambda i: (0, 0)),
            pl.BlockSpec((1, D_IN), lambda i: (0, 0)),
        ],
        out_specs=pl.BlockSpec((blk, D_IN), lambda i: (i, 0)),
        out_shape=jax.ShapeDtypeStruct((N_NODES, D_IN), jnp.float32),
    )(x, agg, deg, wlt, wrt, b2)


@jax.jit
def kernel(x, edge_index, W_l, W_r, b):
    src4 = edge_index[0].astype(jnp.int32).reshape(NC, NS, NSTEPS, CHUNK)
    dst4 = edge_index[1].astype(jnp.int32).reshape(NC, NS, NSTEPS, CHUNK)
    zf = jnp.zeros((N_NODES, D_IN), jnp.float32)
    zd = jnp.zeros((N_NODES, DEG_W), jnp.float32)
    ones = jnp.ones((CHUNK, DEG_W), jnp.float32)
    agg, deg = _sc_aggregate(x, src4, dst4, zf, zd, ones)
    return _tc_combine(x, agg, deg, W_l.T, W_r.T, b[None, :])
